# baseline (speedup 1.0000x reference)
import functools

import jax
import jax.numpy as jnp
from jax import lax
from jax.experimental import pallas as pl
from jax.experimental.pallas import tpu as pltpu
from jax.experimental.pallas import tpu_sc as plsc

N = 10000
NP = 10240
E = 320000
D = 128
H = 16
C = 40
CP = 48
NC = 2
NS = 16
NW = NC * NS
EPW = E // NW
BLK = 80
NBLK = EPW // BLK
RPT = NP // NS

_f32 = jnp.float32


def _mesh():
    return plsc.VectorSubcoreMesh(core_axis_name="c", subcore_axis_name="s")


_SC_PARAMS = pltpu.CompilerParams(
    needs_layout_passes=False, use_tc_tiling_on_sc=False
)


def _sc_degrees(edge_index):
    @functools.partial(
        pl.kernel,
        out_type=jax.ShapeDtypeStruct((2 * NW, NP), _f32),
        mesh=_mesh(),
        compiler_params=_SC_PARAMS,
        scratch_types=[
            pltpu.VMEM((EPW,), jnp.int32),
            pltpu.VMEM((EPW,), jnp.int32),
            pltpu.VMEM((NP,), _f32),
            pltpu.VMEM((NP,), _f32),
        ],
    )
    def k(e_hbm, out_hbm, sidx, didx, deg_o, deg_i):
        cid = lax.axis_index("c")
        sid = lax.axis_index("s")
        wid = cid * NS + sid
        pltpu.sync_copy(e_hbm.at[0].at[pl.ds(wid * EPW, EPW)], sidx)
        pltpu.sync_copy(e_hbm.at[1].at[pl.ds(wid * EPW, EPW)], didx)

        zeros16 = jnp.zeros((16,), _f32)

        @pl.loop(0, NP // 16)
        def _(i):
            deg_o[pl.ds(i * 16, 16)] = zeros16
            deg_i[pl.ds(i * 16, 16)] = zeros16

        ones16 = jnp.ones((16,), _f32)

        @pl.loop(0, EPW // 16)
        def _(i):
            plsc.addupdate_scatter(deg_o, [sidx[pl.ds(i * 16, 16)]], ones16)
            plsc.addupdate_scatter(deg_i, [didx[pl.ds(i * 16, 16)]], ones16)

        pltpu.sync_copy(deg_o, out_hbm.at[wid])
        pltpu.sync_copy(deg_i, out_hbm.at[NW + wid])

    return k(edge_index)


def _sc_scatter(h, edge_index, F):
    BLKW = 128
    EPWP = 10240
    NBLKW = EPWP // BLKW
    NHALF = 3
    NSET = 2 * NHALF
    ROUNDS = NBLKW // NSET
    TAIL = NBLKW - ROUNDS * NSET

    @functools.partial(
        pl.kernel,
        out_type=jax.ShapeDtypeStruct((NC, NP, F), _f32),
        mesh=_mesh(),
        compiler_params=_SC_PARAMS,
        scratch_types=[
            pltpu.VMEM((EPWP,), jnp.int32),
            pltpu.VMEM((EPWP,), jnp.int32),
            pltpu.VMEM((NSET, BLKW, F), _f32),
            pltpu.VMEM((RPT, F), _f32),
            pltpu.VMEM_SHARED((NP, F), _f32),
            pltpu.SemaphoreType.DMA((NSET,)),
            pltpu.SemaphoreType.DMA((NSET,)),
        ],
    )
    def k(h_hbm, e_hbm, out_hbm, sidx, didx, rows, stage, agg, gsem, ssem):
        cid = lax.axis_index("c")
        sid = lax.axis_index("s")
        wid = cid * NS + sid
        pltpu.sync_copy(e_hbm.at[0].at[pl.ds(wid * EPW, EPW)],
                        sidx.at[pl.ds(0, EPW)])
        pltpu.sync_copy(e_hbm.at[1].at[pl.ds(wid * EPW, EPW)],
                        didx.at[pl.ds(0, EPW)])
        dummy = jnp.full((16,), NP - 1, jnp.int32)
        for t in range(EPW // 16, EPWP // 16):
            sidx[pl.ds(t * 16, 16)] = dummy
            didx[pl.ds(t * 16, 16)] = dummy

        def gather(b, p):
            pltpu.async_copy(
                h_hbm.at[sidx.at[pl.ds(b * BLKW, BLKW)]], rows.at[p],
                gsem.at[p]
            )

        def gather_wait(p):
            pltpu.make_async_copy(
                h_hbm.at[pl.ds(0, BLKW)], rows.at[p], gsem.at[p]
            ).wait()

        def scatter(b, p):
            pltpu.async_copy(
                rows.at[p], agg.at[didx.at[pl.ds(b * BLKW, BLKW)]],
                ssem.at[p], add=True,
            )

        def scatter_wait(p):
            pltpu.make_async_copy(
                rows.at[p], agg.at[pl.ds(0, BLKW)], ssem.at[p]
            ).wait()

        zeros16 = jnp.zeros((16,), _f32)

        @pl.loop(0, RPT)
        def _(r):
            row = stage.at[r]

            @pl.loop(0, F // 16)
            def _(v):
                row[pl.ds(v * 16, 16)] = zeros16

        pltpu.sync_copy(stage, agg.at[pl.ds(sid * RPT, RPT)])
        plsc.subcore_barrier()

        for p in range(NSET):
            gather(p, p)

        @pl.loop(0, ROUNDS)
        def _(g):
            base = g * NSET
            for p in range(NHALF):
                gather_wait(p)
                scatter(base + p, p)
            for p in range(NHALF, NSET):
                gather_wait(p)
                scatter(base + p, p)
            for p in range(NHALF):
                nb = base + NSET + p
                scatter_wait(p)

                @pl.when(nb < NBLKW)
                def _():
                    gather(nb, p)

            for p in range(NHALF, NSET):
                nb = base + NSET + p
                scatter_wait(p)

                @pl.when(nb < NBLKW)
                def _():
                    gather(nb, p)

        for p in range(TAIL):
            gather_wait(p)
            scatter(ROUNDS * NSET + p, p)
        for p in range(TAIL):
            scatter_wait(p)

        plsc.subcore_barrier()
        pltpu.sync_copy(agg.at[pl.ds(sid * RPT, RPT)], stage)
        pltpu.sync_copy(stage, out_hbm.at[cid].at[pl.ds(sid * RPT, RPT)])

    return k(h, edge_index)


def _tc0(x_p, W1):
    def body(x_ref, w_ref, h_ref):
        h_ref[...] = lax.dot_general(
            w_ref[...], x_ref[...],
            dimension_numbers=(((0,), (1,)), ((), ())),
            preferred_element_type=_f32,
        )

    return pl.pallas_call(
        body,
        out_shape=jax.ShapeDtypeStruct((H, NP), _f32),
    )(x_p, W1)


def _tc1(deg_parts, h1T):
    def body(deg_ref, h_ref, norms_ref, hs_ref):
        deg = deg_ref[...]
        deg_o = jnp.sum(deg[:NW], axis=0, keepdims=True)
        deg_i = jnp.sum(deg[NW:], axis=0, keepdims=True)
        ns = lax.rsqrt(jnp.maximum(deg_o, 1.0))
        nd = lax.rsqrt(jnp.maximum(deg_i, 1.0))
        norms_ref[...] = jnp.concatenate([ns, nd], axis=0)
        hs_ref[...] = h_ref[...] * ns

    return pl.pallas_call(
        body,
        out_shape=(
            jax.ShapeDtypeStruct((2, NP), _f32),
            jax.ShapeDtypeStruct((H, NP), _f32),
        ),
    )(deg_parts, h1T)


def _tc2(agg1T, norms, b1c, W2):
    def body(a_ref, n_ref, b_ref, w_ref, out_ref):
        ns = n_ref[0:1, :]
        nd = n_ref[1:2, :]
        agg = a_ref[0] + a_ref[1]
        h = jnp.maximum(agg * nd + b_ref[...], 0.0)
        h2 = lax.dot_general(
            w_ref[...], h,
            dimension_numbers=(((0,), (0,)), ((), ())),
            preferred_element_type=_f32,
        )
        out_ref[:C, :] = h2 * ns
        out_ref[C:, :] = jnp.zeros((CP - C, NP), _f32)

    return pl.pallas_call(
        body,
        out_shape=jax.ShapeDtypeStruct((CP, NP), _f32),
    )(agg1T, norms, b1c, W2)


def _tc3(agg2T, norms, b2c):
    def body(a_ref, n_ref, b_ref, out_ref):
        nd = n_ref[1:2, :]
        z = (a_ref[0] + a_ref[1])[:C, :] * nd + b_ref[...]
        m = jnp.max(z, axis=0, keepdims=True)
        e = jnp.exp(z - m)
        lse = jnp.log(jnp.sum(e, axis=0, keepdims=True)) + m
        out_ref[...] = z - lse

    return pl.pallas_call(
        body,
        out_shape=jax.ShapeDtypeStruct((C, NP), _f32),
    )(agg2T, norms, b2c)


def kernel(x, edge_index, W1, b1, W2, b2):
    x_p = jnp.pad(x, ((0, NP - N), (0, 0)))
    b1c = b1[:, None]
    b2c = b2[:, None]

    deg_parts = _sc_degrees(edge_index)
    h1T = _tc0(x_p, W1)
    norms, h1sT = _tc1(deg_parts, h1T)
    agg1 = _sc_scatter(h1sT.T, edge_index, H)
    agg1T = agg1.transpose(0, 2, 1)
    h2sT = _tc2(agg1T, norms, b1c, W2)
    agg2 = _sc_scatter(h2sT.T, edge_index, CP)
    agg2T = agg2.transpose(0, 2, 1)
    outT = _tc3(agg2T, norms, b2c)
    return outT.T[:N]


# speedup vs baseline: 1.8015x; 1.8015x over previous
import functools

import jax
import jax.numpy as jnp
from jax import lax
from jax.experimental import pallas as pl
from jax.experimental.pallas import tpu as pltpu
from jax.experimental.pallas import tpu_sc as plsc

N = 10000
NP = 10240
E = 320000
D = 128
H = 16
C = 40
CP = 48
NC = 2
NS = 16
NW = NC * NS
EPW = E // NW
BLK = 80
NBLK = EPW // BLK
RPT = NP // NS

_f32 = jnp.float32


def _mesh():
    return plsc.VectorSubcoreMesh(core_axis_name="c", subcore_axis_name="s")


_SC_PARAMS = pltpu.CompilerParams(
    needs_layout_passes=False, use_tc_tiling_on_sc=False
)


def _sc_degrees(edge_index):
    @functools.partial(
        pl.kernel,
        out_type=jax.ShapeDtypeStruct((2 * NW, NP), _f32),
        mesh=_mesh(),
        compiler_params=_SC_PARAMS,
        scratch_types=[
            pltpu.VMEM((EPW,), jnp.int32),
            pltpu.VMEM((EPW,), jnp.int32),
            pltpu.VMEM((NP,), _f32),
            pltpu.VMEM((NP,), _f32),
        ],
    )
    def k(e_hbm, out_hbm, sidx, didx, deg_o, deg_i):
        cid = lax.axis_index("c")
        sid = lax.axis_index("s")
        wid = cid * NS + sid
        pltpu.sync_copy(e_hbm.at[0].at[pl.ds(wid * EPW, EPW)], sidx)
        pltpu.sync_copy(e_hbm.at[1].at[pl.ds(wid * EPW, EPW)], didx)

        zeros16 = jnp.zeros((16,), _f32)

        @pl.loop(0, NP // 16)
        def _(i):
            deg_o[pl.ds(i * 16, 16)] = zeros16
            deg_i[pl.ds(i * 16, 16)] = zeros16

        ones16 = jnp.ones((16,), _f32)

        @pl.loop(0, EPW // 80)
        def _(i):
            for u in range(5):
                o = (i * 5 + u) * 16
                plsc.addupdate_scatter(deg_o, [sidx[pl.ds(o, 16)]], ones16)
                plsc.addupdate_scatter(deg_i, [didx[pl.ds(o, 16)]], ones16)

        pltpu.sync_copy(deg_o, out_hbm.at[wid])
        pltpu.sync_copy(deg_i, out_hbm.at[NW + wid])

    return k(edge_index)


def _sc_scatter(h, edge_index, F):
    NHALF = 5
    NSET = 2 * NHALF
    ROUNDS = NBLK // NSET
    TAIL = NBLK - ROUNDS * NSET

    @functools.partial(
        pl.kernel,
        out_type=jax.ShapeDtypeStruct((NC, NP, F), _f32),
        mesh=_mesh(),
        compiler_params=_SC_PARAMS,
        scratch_types=[
            pltpu.VMEM((EPW,), jnp.int32),
            pltpu.VMEM((EPW,), jnp.int32),
            pltpu.VMEM((NSET, BLK, F), _f32),
            pltpu.VMEM((RPT, F), _f32),
            pltpu.VMEM_SHARED((NP, F), _f32),
            pltpu.SemaphoreType.DMA((NSET,)),
            pltpu.SemaphoreType.DMA((NSET,)),
        ],
    )
    def k(h_hbm, e_hbm, out_hbm, sidx, didx, rows, stage, agg, gsem, ssem):
        cid = lax.axis_index("c")
        sid = lax.axis_index("s")
        wid = cid * NS + sid
        pltpu.sync_copy(e_hbm.at[0].at[pl.ds(wid * EPW, EPW)], sidx)
        pltpu.sync_copy(e_hbm.at[1].at[pl.ds(wid * EPW, EPW)], didx)

        def gather(b, p):
            pltpu.async_copy(
                h_hbm.at[sidx.at[pl.ds(b * BLK, BLK)]], rows.at[p], gsem.at[p]
            )

        def gather_wait(p):
            pltpu.make_async_copy(
                h_hbm.at[pl.ds(0, BLK)], rows.at[p], gsem.at[p]
            ).wait()

        def scatter(b, p):
            pltpu.async_copy(
                rows.at[p], agg.at[didx.at[pl.ds(b * BLK, BLK)]], ssem.at[p],
                add=True,
            )

        def scatter_wait(p):
            pltpu.make_async_copy(
                rows.at[p], agg.at[pl.ds(0, BLK)], ssem.at[p]
            ).wait()

        zeros16 = jnp.zeros((16,), _f32)

        @pl.loop(0, RPT)
        def _(r):
            row = stage.at[r]

            @pl.loop(0, F // 16)
            def _(v):
                row[pl.ds(v * 16, 16)] = zeros16

        pltpu.sync_copy(stage, agg.at[pl.ds(sid * RPT, RPT)])
        plsc.subcore_barrier()

        for p in range(NSET):
            gather(p, p)

        @pl.loop(0, ROUNDS)
        def _(g):
            base = g * NSET
            for p in range(NHALF):
                gather_wait(p)
                scatter(base + p, p)
            for p in range(NHALF, NSET):
                gather_wait(p)
                scatter(base + p, p)
            for p in range(NHALF):
                nb = base + NSET + p
                scatter_wait(p)

                @pl.when(nb < NBLK)
                def _():
                    gather(nb, p)

            for p in range(NHALF, NSET):
                nb = base + NSET + p
                scatter_wait(p)

                @pl.when(nb < NBLK)
                def _():
                    gather(nb, p)

        for p in range(TAIL):
            gather_wait(p)
            scatter(ROUNDS * NSET + p, p)
        for p in range(TAIL):
            scatter_wait(p)

        plsc.subcore_barrier()
        pltpu.sync_copy(agg.at[pl.ds(sid * RPT, RPT)], stage)
        pltpu.sync_copy(stage, out_hbm.at[cid].at[pl.ds(sid * RPT, RPT)])

    return k(h, edge_index)


def _tc0(x_p, W1):
    def body(x_ref, w_ref, h_ref):
        h_ref[...] = lax.dot_general(
            w_ref[...], x_ref[...],
            dimension_numbers=(((0,), (1,)), ((), ())),
            preferred_element_type=_f32,
        )

    return pl.pallas_call(
        body,
        out_shape=jax.ShapeDtypeStruct((H, NP), _f32),
    )(x_p, W1)


def _tc1(deg_parts, h1T):
    def body(deg_ref, h_ref, norms_ref, hs_ref):
        deg = deg_ref[...]
        deg_o = jnp.sum(deg[:NW], axis=0, keepdims=True)
        deg_i = jnp.sum(deg[NW:], axis=0, keepdims=True)
        ns = lax.rsqrt(jnp.maximum(deg_o, 1.0))
        nd = lax.rsqrt(jnp.maximum(deg_i, 1.0))
        norms_ref[...] = jnp.concatenate([ns, nd], axis=0)
        hs_ref[...] = h_ref[...] * ns

    return pl.pallas_call(
        body,
        out_shape=(
            jax.ShapeDtypeStruct((2, NP), _f32),
            jax.ShapeDtypeStruct((H, NP), _f32),
        ),
    )(deg_parts, h1T)


def _tc2(agg1T, norms, b1c, W2):
    def body(a_ref, n_ref, b_ref, w_ref, out_ref):
        ns = n_ref[0:1, :]
        nd = n_ref[1:2, :]
        agg = a_ref[0] + a_ref[1]
        h = jnp.maximum(agg * nd + b_ref[...], 0.0)
        h2 = lax.dot_general(
            w_ref[...], h,
            dimension_numbers=(((0,), (0,)), ((), ())),
            preferred_element_type=_f32,
        )
        out_ref[:C, :] = h2 * ns
        out_ref[C:, :] = jnp.zeros((CP - C, NP), _f32)

    return pl.pallas_call(
        body,
        out_shape=jax.ShapeDtypeStruct((CP, NP), _f32),
    )(agg1T, norms, b1c, W2)


def _tc3(agg2T, norms, b2c):
    def body(a_ref, n_ref, b_ref, out_ref):
        nd = n_ref[1:2, :]
        z = (a_ref[0] + a_ref[1])[:C, :] * nd + b_ref[...]
        m = jnp.max(z, axis=0, keepdims=True)
        e = jnp.exp(z - m)
        lse = jnp.log(jnp.sum(e, axis=0, keepdims=True)) + m
        out_ref[...] = z - lse

    return pl.pallas_call(
        body,
        out_shape=jax.ShapeDtypeStruct((C, NP), _f32),
    )(agg2T, norms, b2c)


def kernel(x, edge_index, W1, b1, W2, b2):
    x_p = jnp.pad(x, ((0, NP - N), (0, 0)))
    b1c = b1[:, None]
    b2c = b2[:, None]

    deg_parts = _sc_degrees(edge_index)
    h1T = _tc0(x_p, W1)
    norms, h1sT = _tc1(deg_parts, h1T)
    agg1 = _sc_scatter(h1sT.T, edge_index, H)
    agg1T = agg1.transpose(0, 2, 1)
    h2sT = _tc2(agg1T, norms, b1c, W2)
    agg2 = _sc_scatter(h2sT.T, edge_index, CP)
    agg2T = agg2.transpose(0, 2, 1)
    outT = _tc3(agg2T, norms, b2c)
    return outT.T[:N]


# speedup vs baseline: 2.0033x; 1.1120x over previous
import functools

import jax
import jax.numpy as jnp
from jax import lax
from jax.experimental import pallas as pl
from jax.experimental.pallas import tpu as pltpu
from jax.experimental.pallas import tpu_sc as plsc

N = 10000
NP = 10240
E = 320000
D = 128
H = 16
C = 40
CP = 48
NC = 2
NS = 16
NW = NC * NS
EPW = E // NW
BLK = 80
NBLK = EPW // BLK
RPT = NP // NS

_f32 = jnp.float32


def _mesh():
    return plsc.VectorSubcoreMesh(core_axis_name="c", subcore_axis_name="s")


_SC_PARAMS = pltpu.CompilerParams(
    needs_layout_passes=False, use_tc_tiling_on_sc=False
)


def _sc_degrees(edge_index):
    @functools.partial(
        pl.kernel,
        out_type=jax.ShapeDtypeStruct((2 * NW, NP), _f32),
        mesh=_mesh(),
        compiler_params=_SC_PARAMS,
        scratch_types=[
            pltpu.VMEM((EPW,), jnp.int32),
            pltpu.VMEM((EPW,), jnp.int32),
            pltpu.VMEM((NP,), _f32),
            pltpu.VMEM((NP,), _f32),
        ],
    )
    def k(e_hbm, out_hbm, sidx, didx, deg_o, deg_i):
        cid = lax.axis_index("c")
        sid = lax.axis_index("s")
        wid = cid * NS + sid
        pltpu.sync_copy(e_hbm.at[0].at[pl.ds(wid * EPW, EPW)], sidx)
        pltpu.sync_copy(e_hbm.at[1].at[pl.ds(wid * EPW, EPW)], didx)

        zeros16 = jnp.zeros((16,), _f32)

        @pl.loop(0, NP // 16)
        def _(i):
            deg_o[pl.ds(i * 16, 16)] = zeros16
            deg_i[pl.ds(i * 16, 16)] = zeros16

        ones16 = jnp.ones((16,), _f32)

        @pl.loop(0, EPW // 80)
        def _(i):
            for u in range(5):
                o = (i * 5 + u) * 16
                plsc.addupdate_scatter(deg_o, [sidx[pl.ds(o, 16)]], ones16)
                plsc.addupdate_scatter(deg_i, [didx[pl.ds(o, 16)]], ones16)

        pltpu.sync_copy(deg_o, out_hbm.at[wid])
        pltpu.sync_copy(deg_i, out_hbm.at[NW + wid])

    return k(edge_index)


def _sc_scatter(h, edge_index, F):
    NHALF = 5
    NSET = 2 * NHALF
    ROUNDS = NBLK // NSET
    TAIL = NBLK - ROUNDS * NSET

    @functools.partial(
        pl.kernel,
        out_type=jax.ShapeDtypeStruct((NC, NP, F), _f32),
        mesh=_mesh(),
        compiler_params=_SC_PARAMS,
        scratch_types=[
            pltpu.VMEM((EPW,), jnp.int32),
            pltpu.VMEM((EPW,), jnp.int32),
            pltpu.VMEM((NSET, BLK, F), _f32),
            pltpu.VMEM((RPT, F), _f32),
            pltpu.VMEM_SHARED((NP, F), _f32),
            pltpu.SemaphoreType.DMA((NSET,)),
            pltpu.SemaphoreType.DMA((NSET,)),
        ],
    )
    def k(h_hbm, e_hbm, out_hbm, sidx, didx, rows, stage, agg, gsem, ssem):
        cid = lax.axis_index("c")
        sid = lax.axis_index("s")
        wid = cid * NS + sid
        pltpu.sync_copy(e_hbm.at[0].at[pl.ds(wid * EPW, EPW)], sidx)
        pltpu.sync_copy(e_hbm.at[1].at[pl.ds(wid * EPW, EPW)], didx)

        def gather(b, p):
            pltpu.async_copy(
                h_hbm.at[sidx.at[pl.ds(b * BLK, BLK)]], rows.at[p], gsem.at[p]
            )

        def gather_wait(p):
            pltpu.make_async_copy(
                h_hbm.at[pl.ds(0, BLK)], rows.at[p], gsem.at[p]
            ).wait()

        def scatter(b, p):
            pltpu.async_copy(
                rows.at[p], agg.at[didx.at[pl.ds(b * BLK, BLK)]], ssem.at[p],
                add=True,
            )

        def scatter_wait(p):
            pltpu.make_async_copy(
                rows.at[p], agg.at[pl.ds(0, BLK)], ssem.at[p]
            ).wait()

        zeros16 = jnp.zeros((16,), _f32)

        @pl.loop(0, RPT)
        def _(r):
            row = stage.at[r]

            @pl.loop(0, F // 16)
            def _(v):
                row[pl.ds(v * 16, 16)] = zeros16

        pltpu.sync_copy(stage, agg.at[pl.ds(sid * RPT, RPT)])
        plsc.subcore_barrier()

        for p in range(NSET):
            gather(p, p)

        @pl.loop(0, ROUNDS)
        def _(g):
            base = g * NSET
            for p in range(NHALF):
                gather_wait(p)
                scatter(base + p, p)
            for p in range(NHALF, NSET):
                gather_wait(p)
                scatter(base + p, p)
            for p in range(NHALF):
                nb = base + NSET + p
                scatter_wait(p)

                @pl.when(nb < NBLK)
                def _():
                    gather(nb, p)

            for p in range(NHALF, NSET):
                nb = base + NSET + p
                scatter_wait(p)

                @pl.when(nb < NBLK)
                def _():
                    gather(nb, p)

        for p in range(TAIL):
            gather_wait(p)
            scatter(ROUNDS * NSET + p, p)
        for p in range(TAIL):
            scatter_wait(p)

        plsc.subcore_barrier()
        pltpu.sync_copy(agg.at[pl.ds(sid * RPT, RPT)], stage)
        pltpu.sync_copy(stage, out_hbm.at[cid].at[pl.ds(sid * RPT, RPT)])

    return k(h, edge_index)


def _tc0(x_p, W1):
    def body(x_ref, w_ref, h_ref):
        h_ref[...] = lax.dot_general(
            w_ref[...], x_ref[...],
            dimension_numbers=(((0,), (1,)), ((), ())),
            preferred_element_type=_f32,
        )

    return pl.pallas_call(
        body,
        out_shape=jax.ShapeDtypeStruct((H, NP), _f32),
    )(x_p, W1)


def _tc1(deg_parts, h1T):
    def body(deg_ref, h_ref, norms_ref, hs_ref):
        deg = deg_ref[...]
        deg_o = jnp.sum(deg[:NW], axis=0, keepdims=True)
        deg_i = jnp.sum(deg[NW:], axis=0, keepdims=True)
        ns = lax.rsqrt(jnp.maximum(deg_o, 1.0))
        nd = lax.rsqrt(jnp.maximum(deg_i, 1.0))
        norms_ref[...] = jnp.concatenate([ns, nd], axis=0)
        hs_ref[...] = h_ref[...] * ns

    return pl.pallas_call(
        body,
        out_shape=(
            jax.ShapeDtypeStruct((2, NP), _f32),
            jax.ShapeDtypeStruct((H, NP), _f32),
        ),
    )(deg_parts, h1T)


def _tc2(agg1p, nd_p16, b1_p, W2bd, ns_p48):
    NR = NP // 8

    def body(a_ref, nd_ref, b_ref, w_ref, ns_ref, out_ref):
        h = jnp.maximum(
            (a_ref[0] + a_ref[1]) * nd_ref[...] + b_ref[...], 0.0
        )
        h2 = jnp.dot(h, w_ref[...], preferred_element_type=_f32)
        out_ref[...] = h2 * ns_ref[...]

    return pl.pallas_call(
        body,
        out_shape=jax.ShapeDtypeStruct((NR, 8 * CP), _f32),
    )(agg1p, nd_p16, b1_p, W2bd, ns_p48)


def _tc3(agg2p, nd_p48, b2_p):
    NR = NP // 8

    def body(a_ref, nd_ref, b_ref, out_ref):
        z = (a_ref[0] + a_ref[1]) * nd_ref[...] + b_ref[...]
        z3 = z.reshape(NR, 8, CP)
        m = jnp.max(z3, axis=-1, keepdims=True)
        e = jnp.exp(z3 - m)
        lse = jnp.log(jnp.sum(e, axis=-1, keepdims=True)) + m
        out_ref[...] = (z3 - lse).reshape(NR, 8 * CP)

    return pl.pallas_call(
        body,
        out_shape=jax.ShapeDtypeStruct((NR, 8 * CP), _f32),
    )(agg2p, nd_p48, b2_p)


def kernel(x, edge_index, W1, b1, W2, b2):
    NR = NP // 8
    x_p = jnp.pad(x, ((0, NP - N), (0, 0)))
    b1_p = jnp.tile(b1, 8)[None, :]
    b2_pad = jnp.pad(b2, (0, CP - C), constant_values=-1e30)
    b2_p = jnp.tile(b2_pad, 8)[None, :]
    W2bd = jnp.kron(jnp.eye(8, dtype=_f32),
                    jnp.pad(W2, ((0, 0), (0, CP - C))))

    deg_parts = _sc_degrees(edge_index)
    h1T = _tc0(x_p, W1)
    norms, h1sT = _tc1(deg_parts, h1T)

    ns = norms[0].reshape(NR, 8, 1)
    nd = norms[1].reshape(NR, 8, 1)
    nd_p16 = jnp.broadcast_to(nd, (NR, 8, H)).reshape(NR, 8 * H)
    ns_p48 = jnp.broadcast_to(ns, (NR, 8, CP)).reshape(NR, 8 * CP)
    nd_p48 = jnp.broadcast_to(nd, (NR, 8, CP)).reshape(NR, 8 * CP)

    agg1 = _sc_scatter(h1sT.T, edge_index, H)
    agg1p = agg1.reshape(NC, NR, 8 * H)
    h2sp = _tc2(agg1p, nd_p16, b1_p, W2bd, ns_p48)
    h2s = h2sp.reshape(NP, CP)
    agg2 = _sc_scatter(h2s, edge_index, CP)
    agg2p = agg2.reshape(NC, NR, 8 * CP)
    outp = _tc3(agg2p, nd_p48, b2_p)
    return outp.reshape(NP, CP)[:N, :C]
